# baseline (speedup 1.0000x reference)
import functools

import jax
import jax.numpy as jnp
from jax import lax
from jax.experimental import pallas as pl
from jax.experimental.pallas import tpu as pltpu
from jax.experimental.pallas import tpu_sc as plsc

B = 4096
F = 26
V = 100000
D = 64
DENSE = 13
H1 = 1024
H2 = 512

_N = B * F
_NC = 2
_NS = 16
_NW = _NC * _NS
_PER_W = _N // _NW
_CHUNK = 128
_NCHUNK = _PER_W // _CHUNK


def _gather_rows(flat_tables, flat_idx):
    mesh = plsc.VectorSubcoreMesh(core_axis_name="c", subcore_axis_name="s")

    @functools.partial(
        pl.kernel,
        out_type=jax.ShapeDtypeStruct((_N, D), jnp.float32),
        mesh=mesh,
        scratch_types=[
            pltpu.VMEM((_PER_W,), jnp.int32),
            pltpu.VMEM((_CHUNK, D), jnp.float32),
            pltpu.SemaphoreType.DMA,
        ],
        compiler_params=pltpu.CompilerParams(use_tc_tiling_on_sc=False),
    )
    def gather_kernel(tab_hbm, idx_hbm, out_hbm, idx_v, buf, sem):
        wid = lax.axis_index("s") * _NC + lax.axis_index("c")
        base = wid * _PER_W
        pltpu.sync_copy(idx_hbm.at[pl.ds(base, _PER_W)], idx_v)

        def body(c, carry):
            off = c * _CHUNK
            pltpu.async_copy(
                tab_hbm.at[idx_v.at[pl.ds(off, _CHUNK)]], buf, sem
            ).wait()
            pltpu.sync_copy(buf, out_hbm.at[pl.ds(base + off, _CHUNK)])
            return carry

        lax.fori_loop(0, _NCHUNK, body, 0)

    return gather_kernel(flat_tables, flat_idx)


_BB = 512


def _mlp_body(emb_ref, dense_ref, w1_ref, b1_ref, w2_ref, b2_ref, w3_ref,
              b3_ref, out_ref):
    h = jnp.dot(emb_ref[...], w1_ref[0:F * D, :],
                preferred_element_type=jnp.float32)
    h = h + jnp.dot(dense_ref[...], w1_ref[F * D:F * D + DENSE, :],
                    preferred_element_type=jnp.float32)
    h = jnp.maximum(h + b1_ref[...], 0.0)
    h2 = jnp.dot(h, w2_ref[...], preferred_element_type=jnp.float32)
    h2 = jnp.maximum(h2 + b2_ref[...], 0.0)
    logit = jnp.sum(h2 * w3_ref[...], axis=1, keepdims=True) + b3_ref[...]
    out_ref[...] = jax.nn.sigmoid(logit)


def _mlp(embf, dense, W1, b1, W2, b2, w3row, b3):
    return pl.pallas_call(
        _mlp_body,
        grid=(B // _BB,),
        in_specs=[
            pl.BlockSpec((_BB, F * D), lambda i: (i, 0)),
            pl.BlockSpec((_BB, DENSE), lambda i: (i, 0)),
            pl.BlockSpec((F * D + DENSE, H1), lambda i: (0, 0)),
            pl.BlockSpec((1, H1), lambda i: (0, 0)),
            pl.BlockSpec((H1, H2), lambda i: (0, 0)),
            pl.BlockSpec((1, H2), lambda i: (0, 0)),
            pl.BlockSpec((1, H2), lambda i: (0, 0)),
            pl.BlockSpec((1, 1), lambda i: (0, 0)),
        ],
        out_specs=pl.BlockSpec((_BB, 1), lambda i: (i, 0)),
        out_shape=jax.ShapeDtypeStruct((B, 1), jnp.float32),
    )(embf, dense, W1, b1, W2, b2, w3row, b3)


def kernel(sparse_ids, dense_feats, tables, W1, b1, W2, b2, W3, b3):
    flat_tables = tables.reshape(F * V, D)
    offs = (jnp.arange(F, dtype=jnp.int32) * V)[None, :]
    flat_idx = (sparse_ids.astype(jnp.int32) + offs).reshape(_N)
    emb = _gather_rows(flat_tables, flat_idx)
    embf = emb.reshape(B, F * D)
    return _mlp(embf, dense_feats, W1, b1.reshape(1, H1), W2,
                b2.reshape(1, H2), W3.reshape(1, H2), b3.reshape(1, 1))
